# baseline (speedup 1.0000x reference)
import functools

import jax
import jax.numpy as jnp
from jax import lax
from jax.experimental import pallas as pl
from jax.experimental.pallas import tpu as pltpu
from jax.experimental.pallas import tpu_sc as plsc

_NC = 2
_NS = 16
_NW = _NC * _NS

_CHUNK = 256
_RING = 3
_LANES = 16


def _subblocks(n):
    out, off = [], 0
    while off < n:
        sz = min(128, n - off)
        out.append((off, sz))
        off += sz
    return out


def _gather_sum_call(E, N, D):
    epw = E // _NW
    n_full = epw // _CHUNK
    tail = epw - n_full * _CHUNK
    n_main = (n_full // _RING) * _RING
    assert tail < _CHUNK and n_full >= 2 and n_full - n_main <= 2
    mesh = plsc.VectorSubcoreMesh(
        core_axis_name="c", subcore_axis_name="s", num_cores=_NC, num_subcores=_NS
    )

    @functools.partial(
        pl.kernel,
        out_type=jax.ShapeDtypeStruct((E, D), jnp.float32),
        mesh=mesh,
        scratch_types=[
            pltpu.VMEM((epw,), jnp.int32),
            pltpu.VMEM((epw,), jnp.int32),
            pltpu.VMEM((_RING, _CHUNK, D), jnp.float32),
            pltpu.SemaphoreType.DMA,
            pltpu.SemaphoreType.DMA,
            pltpu.SemaphoreType.DMA,
            pltpu.SemaphoreType.DMA,
            pltpu.SemaphoreType.DMA,
            pltpu.SemaphoreType.DMA,
            pltpu.SemaphoreType.DMA,
            pltpu.SemaphoreType.DMA,
            pltpu.SemaphoreType.DMA,
        ],
    )
    def gather_sum(
        x_hbm, s_hbm, r_hbm, g_hbm, idxs, idxr, rows,
        s0, s1, s2, a0, a1, a2, w0, w1, w2,
    ):
        wid = lax.axis_index("s") * _NC + lax.axis_index("c")
        w_base = wid * epw
        ssem = (s0, s1, s2)
        asem = (a0, a1, a2)
        wsem = (w0, w1, w2)

        pltpu.sync_copy(s_hbm.at[pl.ds(w_base, epw)], idxs)
        pltpu.sync_copy(r_hbm.at[pl.ds(w_base, epw)], idxr)

        def gs(ci, p, n=_CHUNK):
            for off, sz in _subblocks(n):
                pltpu.async_copy(
                    x_hbm.at[idxs.at[pl.ds(ci * _CHUNK + off, sz)]],
                    rows.at[p, pl.ds(off, sz)],
                    ssem[p],
                )

        def wait_gs(p, n=_CHUNK):
            pltpu.make_async_copy(
                x_hbm.at[idxs.at[pl.ds(0, n)]], rows.at[p, pl.ds(0, n)], ssem[p]
            ).wait()

        def ga(ci, p, n=_CHUNK):
            for off, sz in _subblocks(n):
                pltpu.async_copy(
                    x_hbm.at[idxr.at[pl.ds(ci * _CHUNK + off, sz)]],
                    rows.at[p, pl.ds(off, sz)],
                    asem[p],
                    add=True,
                )

        def wait_ga(p, n=_CHUNK):
            pltpu.make_async_copy(
                x_hbm.at[idxr.at[pl.ds(0, n)]], rows.at[p, pl.ds(0, n)], asem[p]
            ).wait()

        def wb(ci, p, n=_CHUNK):
            pltpu.async_copy(
                rows.at[p, pl.ds(0, n)],
                g_hbm.at[pl.ds(w_base + ci * _CHUNK, n)],
                wsem[p],
            )

        def wait_wb(p, n=_CHUNK):
            pltpu.make_async_copy(
                rows.at[p, pl.ds(0, n)], g_hbm.at[pl.ds(w_base, n)], wsem[p]
            ).wait()

        gs(0, 0)
        gs(1, 1)
        wait_gs(0)
        ga(0, 0)

        def super_body(j, carry):
            for b in range(_RING):
                i = _RING * j + b
                p1 = (b + 1) % _RING
                p2 = (b + 2) % _RING

                @pl.when(i + 1 < n_full)
                def _():
                    wait_gs(p1)
                    ga(i + 1, p1)

                @pl.when((i >= 1) & (i + 2 < n_full))
                def _():
                    wait_wb(p2)

                @pl.when(i + 2 < n_full)
                def _():
                    gs(i + 2, p2)

                wait_ga(b)
                wb(i, b)
            return carry

        lax.fori_loop(0, n_main // _RING, super_body, 0)

        waited_hi = min(n_main - 2, n_full - 4)
        pending_wb = set(range(waited_hi + 1, n_main))
        for ci in range(n_main, n_full):
            p = ci % _RING
            if ci > n_main:
                wait_gs(p)
                ga(ci, p)
            wait_ga(p)
            wb(ci, p)
            pending_wb.add(ci)
        if tail:
            p = n_full % _RING
            prev = n_full - _RING
            if prev in pending_wb:
                wait_wb(p)
                pending_wb.discard(prev)
            gs(n_full, p, tail)
            wait_gs(p, tail)
            ga(n_full, p, tail)
            wait_ga(p, tail)
            wb(n_full, p, tail)
            wait_wb(p, tail)
        for ci in sorted(pending_wb):
            wait_wb(ci % _RING)

    return gather_sum


def _combine_slab(g_slab, ea_slab, W, b2d, t_prev, E, off_blocks, block_e):
    Es, D = g_slab.shape
    K = ea_slab.shape[1]

    if t_prev is None:
        def body(g_ref, ea_ref, w_ref, b_ref, o_ref):
            proj = (
                jnp.dot(ea_ref[...], w_ref[...], preferred_element_type=jnp.float32)
                + b_ref[...]
            )
            o_ref[...] = g_ref[...] * proj
        extra_in, extra_specs, aliases = (), (), {}
    else:
        def body(g_ref, ea_ref, w_ref, b_ref, t_ref, o_ref):
            del t_ref
            proj = (
                jnp.dot(ea_ref[...], w_ref[...], preferred_element_type=jnp.float32)
                + b_ref[...]
            )
            o_ref[...] = g_ref[...] * proj
        extra_in = (t_prev,)
        extra_specs = (pl.BlockSpec(memory_space=pl.ANY),)
        aliases = {4: 0}

    return pl.pallas_call(
        body,
        grid=(Es // block_e,),
        in_specs=[
            pl.BlockSpec((block_e, D), lambda i: (i, 0)),
            pl.BlockSpec((block_e, K), lambda i: (i, 0)),
            pl.BlockSpec((K, D), lambda i: (0, 0)),
            pl.BlockSpec((1, D), lambda i: (0, 0)),
            *extra_specs,
        ],
        out_specs=pl.BlockSpec((block_e, D), lambda i: (i + off_blocks, 0)),
        out_shape=jax.ShapeDtypeStruct((E, D), jnp.float32),
        input_output_aliases=aliases,
    )(g_slab, ea_slab, W, b2d, *extra_in)


_SLABS = (160000, 160000)
_BLOCKS = (16000, 16000)


def kernel(senders, receivers, edge_attr, x, W, b):
    E = senders.shape[0]
    N, D = x.shape
    senders = senders.astype(jnp.int32)
    receivers = receivers.astype(jnp.int32)
    b2d = b.reshape(1, D)

    assert sum(_SLABS) == E
    offs = [0]
    for n in _SLABS:
        offs.append(offs[-1] + n)

    gs = [
        _gather_sum_call(Es, N, D)(
            x, senders[offs[s]:offs[s + 1]], receivers[offs[s]:offs[s + 1]]
        )
        for s, Es in enumerate(_SLABS)
    ]
    t = None
    for s, Es in enumerate(_SLABS):
        blk = _BLOCKS[s]
        assert offs[s] % blk == 0 and Es % blk == 0
        t = _combine_slab(
            gs[s], edge_attr[offs[s]:offs[s + 1]], W, b2d, t,
            E, offs[s] // blk, blk,
        )
    return t


# speedup vs baseline: 1.0037x; 1.0037x over previous
import functools

import jax
import jax.numpy as jnp
from jax import lax
from jax.experimental import pallas as pl
from jax.experimental.pallas import tpu as pltpu
from jax.experimental.pallas import tpu_sc as plsc

_NC = 2
_NS = 16
_NW = _NC * _NS

_CHUNK = 128
_RING = 3
_LANES = 16


def _gather_sum_call(E, N, D):
    epw = E // _NW
    n_full = epw // _CHUNK
    tail = epw - n_full * _CHUNK
    assert n_full % _RING == 0 and tail < _CHUNK
    mesh = plsc.VectorSubcoreMesh(
        core_axis_name="c", subcore_axis_name="s", num_cores=_NC, num_subcores=_NS
    )

    @functools.partial(
        pl.kernel,
        out_type=jax.ShapeDtypeStruct((E, D), jnp.float32),
        mesh=mesh,
        scratch_types=[
            pltpu.VMEM((epw,), jnp.int32),
            pltpu.VMEM((epw,), jnp.int32),
            pltpu.VMEM((_RING, _CHUNK, D), jnp.float32),
            pltpu.SemaphoreType.DMA,
            pltpu.SemaphoreType.DMA,
            pltpu.SemaphoreType.DMA,
            pltpu.SemaphoreType.DMA,
            pltpu.SemaphoreType.DMA,
            pltpu.SemaphoreType.DMA,
            pltpu.SemaphoreType.DMA,
            pltpu.SemaphoreType.DMA,
            pltpu.SemaphoreType.DMA,
        ],
    )
    def gather_sum(
        x_hbm, s_hbm, r_hbm, g_hbm, idxs, idxr, rows,
        s0, s1, s2, a0, a1, a2, w0, w1, w2,
    ):
        wid = lax.axis_index("s") * _NC + lax.axis_index("c")
        w_base = wid * epw
        ssem = (s0, s1, s2)
        asem = (a0, a1, a2)
        wsem = (w0, w1, w2)

        pltpu.sync_copy(s_hbm.at[pl.ds(w_base, epw)], idxs)
        pltpu.sync_copy(r_hbm.at[pl.ds(w_base, epw)], idxr)

        def gs(ci, p, n=_CHUNK):
            pltpu.async_copy(
                x_hbm.at[idxs.at[pl.ds(ci * _CHUNK, n)]],
                rows.at[p, pl.ds(0, n)],
                ssem[p],
            )

        def wait_gs(p, n=_CHUNK):
            pltpu.make_async_copy(
                x_hbm.at[idxs.at[pl.ds(0, n)]], rows.at[p, pl.ds(0, n)], ssem[p]
            ).wait()

        def ga(ci, p, n=_CHUNK):
            pltpu.async_copy(
                x_hbm.at[idxr.at[pl.ds(ci * _CHUNK, n)]],
                rows.at[p, pl.ds(0, n)],
                asem[p],
                add=True,
            )

        def wait_ga(p, n=_CHUNK):
            pltpu.make_async_copy(
                x_hbm.at[idxr.at[pl.ds(0, n)]], rows.at[p, pl.ds(0, n)], asem[p]
            ).wait()

        def wb(ci, p, n=_CHUNK):
            pltpu.async_copy(
                rows.at[p, pl.ds(0, n)],
                g_hbm.at[pl.ds(w_base + ci * _CHUNK, n)],
                wsem[p],
            )

        def wait_wb(p, n=_CHUNK):
            pltpu.make_async_copy(
                rows.at[p, pl.ds(0, n)], g_hbm.at[pl.ds(w_base, n)], wsem[p]
            ).wait()

        gs(0, 0)
        gs(1, 1)
        wait_gs(0)
        ga(0, 0)

        def super_body(j, carry):
            for b in range(_RING):
                i = _RING * j + b
                p1 = (b + 1) % _RING
                p2 = (b + 2) % _RING

                @pl.when(i + 1 < n_full)
                def _():
                    wait_gs(p1)
                    ga(i + 1, p1)

                @pl.when((i >= 1) & (i + 2 < n_full))
                def _():
                    wait_wb(p2)

                @pl.when(i + 2 < n_full)
                def _():
                    gs(i + 2, p2)

                wait_ga(b)
                wb(i, b)
            return carry

        lax.fori_loop(0, n_full // _RING, super_body, 0)
        wait_wb((n_full - 3) % _RING)
        wait_wb((n_full - 2) % _RING)
        if tail:
            gs(n_full, 0, tail)
            wait_gs(0, tail)
            ga(n_full, 0, tail)
            wait_ga(0, tail)
            wb(n_full, 0, tail)
            wait_wb(0, tail)
        wait_wb((n_full - 1) % _RING)

    return gather_sum


def _combine_slab(g_slab, ea_slab, W, b2d, t_prev, E, off_blocks, block_e):
    Es, D = g_slab.shape
    K = ea_slab.shape[1]

    if t_prev is None:
        def body(g_ref, ea_ref, w_ref, b_ref, o_ref):
            proj = (
                jnp.dot(ea_ref[...], w_ref[...], preferred_element_type=jnp.float32)
                + b_ref[...]
            )
            o_ref[...] = g_ref[...] * proj
        extra_in, extra_specs, aliases = (), (), {}
    else:
        def body(g_ref, ea_ref, w_ref, b_ref, t_ref, o_ref):
            del t_ref
            proj = (
                jnp.dot(ea_ref[...], w_ref[...], preferred_element_type=jnp.float32)
                + b_ref[...]
            )
            o_ref[...] = g_ref[...] * proj
        extra_in = (t_prev,)
        extra_specs = (pl.BlockSpec(memory_space=pl.ANY),)
        aliases = {4: 0}

    return pl.pallas_call(
        body,
        grid=(Es // block_e,),
        in_specs=[
            pl.BlockSpec((block_e, D), lambda i: (i, 0)),
            pl.BlockSpec((block_e, K), lambda i: (i, 0)),
            pl.BlockSpec((K, D), lambda i: (0, 0)),
            pl.BlockSpec((1, D), lambda i: (0, 0)),
            *extra_specs,
        ],
        out_specs=pl.BlockSpec((block_e, D), lambda i: (i + off_blocks, 0)),
        out_shape=jax.ShapeDtypeStruct((E, D), jnp.float32),
        input_output_aliases=aliases,
    )(g_slab, ea_slab, W, b2d, *extra_in)


_SLABS = (160000, 160000)
_BLOCKS = (8000, 8000)


def kernel(senders, receivers, edge_attr, x, W, b):
    E = senders.shape[0]
    N, D = x.shape
    senders = senders.astype(jnp.int32)
    receivers = receivers.astype(jnp.int32)
    b2d = b.reshape(1, D)

    assert sum(_SLABS) == E
    offs = [0]
    for n in _SLABS:
        offs.append(offs[-1] + n)

    gs = [
        _gather_sum_call(Es, N, D)(
            x, senders[offs[s]:offs[s + 1]], receivers[offs[s]:offs[s + 1]]
        )
        for s, Es in enumerate(_SLABS)
    ]
    t = None
    for s, Es in enumerate(_SLABS):
        blk = _BLOCKS[s]
        assert offs[s] % blk == 0 and Es % blk == 0
        t = _combine_slab(
            gs[s], edge_attr[offs[s]:offs[s + 1]], W, b2d, t,
            E, offs[s] // blk, blk,
        )
    return t
